# baseline (speedup 1.0000x reference)
import functools
from math import sqrt

import jax
import jax.numpy as jnp
from jax import lax
from jax.experimental import pallas as pl
from jax.experimental.pallas import tpu as pltpu
from jax.experimental.pallas import tpu_sc as plsc

L = 16
NC, NS = 2, 16
NW = NC * NS
CH = 16


def _build(B, S, V, D):
    P = S // NW
    NCH = P // CH
    NV = D // L
    scale = float(sqrt(D))
    mesh = plsc.VectorSubcoreMesh(core_axis_name="c", subcore_axis_name="s")

    @functools.partial(
        pl.kernel,
        out_type=jax.ShapeDtypeStruct((B * S, D), jnp.float32),
        mesh=mesh,
        scratch_types=[
            pltpu.VMEM((CH,), jnp.int32),
            pltpu.VMEM((P, D), jnp.float32),
            pltpu.VMEM((CH, D), jnp.float32),
            pltpu.SemaphoreType.DMA,
        ],
    )
    def k(xf_hbm, table_hbm, pe_hbm, out_hbm, idx_v, pe_v, gbuf, sem):
        wid = lax.axis_index("s") * NC + lax.axis_index("c")
        pos_base = wid * P
        pltpu.sync_copy(pe_hbm.at[pl.ds(pos_base, P)], pe_v)
        for b in range(B):
            row_base = b * S + pos_base
            for c in range(NCH):
                pltpu.sync_copy(xf_hbm.at[pl.ds(row_base + c * CH, CH)], idx_v)
                pltpu.async_copy(table_hbm.at[idx_v], gbuf, sem).wait()

                def body(j, _, c=c):
                    sl = pl.ds(j * L, L)
                    for r in range(CH):
                        gbuf[r, sl] = gbuf[r, sl] * scale + pe_v[c * CH + r, sl]
                    return _

                lax.fori_loop(0, NV, body, 0)
                pltpu.sync_copy(gbuf, out_hbm.at[pl.ds(row_base + c * CH, CH)])

    return k


def kernel(x, embed_table, pe):
    B, S = x.shape
    V, D = embed_table.shape
    xf = x.reshape(B * S).astype(jnp.int32)
    pe2 = pe.reshape(pe.shape[-2], pe.shape[-1])[:S]
    out = _build(B, S, V, D)(xf, embed_table, pe2)
    return out.reshape(B, S, D)


# speedup vs baseline: 1.4304x; 1.4304x over previous
import functools
from math import sqrt

import jax
import jax.numpy as jnp
from jax import lax
from jax.experimental import pallas as pl
from jax.experimental.pallas import tpu as pltpu
from jax.experimental.pallas import tpu_sc as plsc

L = 16
NC, NS = 2, 16
NW = NC * NS
CH = 16
NBUF = 4


def _build(B, S, V, D):
    P = S // NW
    NCH = P // CH
    NV = D // L
    T = NCH * B
    scale = float(sqrt(D))
    mesh = plsc.VectorSubcoreMesh(core_axis_name="c", subcore_axis_name="s")

    @functools.partial(
        pl.kernel,
        out_type=jax.ShapeDtypeStruct((B * S, D), jnp.float32),
        mesh=mesh,
        scratch_types=[
            [pltpu.VMEM((CH,), jnp.int32) for _ in range(NBUF)],
            [pltpu.VMEM((CH, D), jnp.float32) for _ in range(2)],
            [pltpu.VMEM((CH, D), jnp.float32) for _ in range(NBUF)],
            pltpu.SemaphoreType.DMA,
            pltpu.SemaphoreType.DMA,
            pltpu.SemaphoreType.DMA,
        ],
    )
    def k(xf_hbm, table_hbm, pe_hbm, out_hbm, idx_v, pe_v, gbuf, gsem, wsem, psem):
        wid = lax.axis_index("s") * NC + lax.axis_index("c")
        pos_base = wid * P

        def row_base(t):
            c, b = divmod(t, B)
            return b * S + pos_base + c * CH

        def start_gather(t):
            ib = idx_v[t % NBUF]
            pltpu.sync_copy(xf_hbm.at[pl.ds(row_base(t), CH)], ib)
            return pltpu.async_copy(table_hbm.at[ib], gbuf[t % NBUF], gsem)

        def start_pe(c):
            return pltpu.async_copy(
                pe_hbm.at[pl.ds(pos_base + c * CH, CH)], pe_v[c % 2], psem)

        pe_copies = [start_pe(0)]
        gathers = [start_gather(0), start_gather(1)]
        writes = []

        for t in range(T):
            c, b = divmod(t, B)
            if b == 0:
                pe_copies[c].wait()
                if c + 1 < NCH:
                    pe_copies.append(start_pe(c + 1))
            gathers[t].wait()
            g = gbuf[t % NBUF]
            p = pe_v[c % 2]

            def body(j, _, g=g, p=p):
                sl = pl.ds(j * L, L)
                for r in range(CH):
                    g[r, sl] = g[r, sl] * scale + p[r, sl]
                return _

            lax.fori_loop(0, NV, body, 0)
            writes.append(pltpu.async_copy(
                g, out_hbm.at[pl.ds(row_base(t), CH)], wsem))
            if t + 2 < T:
                if t - 2 >= 0:
                    writes[t - 2].wait()
                gathers.append(start_gather(t + 2))
        for t in range(max(0, T - 4), T):
            writes[t].wait()

    return k


def kernel(x, embed_table, pe):
    B, S = x.shape
    V, D = embed_table.shape
    xf = x.reshape(B * S).astype(jnp.int32)
    pe2 = pe.reshape(pe.shape[-2], pe.shape[-1])[:S]
    out = _build(B, S, V, D)(xf, embed_table, pe2)
    return out.reshape(B, S, D)


# speedup vs baseline: 1.5691x; 1.0970x over previous
import functools
from math import sqrt

import jax
import jax.numpy as jnp
from jax import lax
from jax.experimental import pallas as pl
from jax.experimental.pallas import tpu as pltpu
from jax.experimental.pallas import tpu_sc as plsc

L = 16
NC, NS = 2, 16
NW = NC * NS
CH = 16
NBUF = 5
LOOKAHEAD = 3


def _build(B, S, V, D):
    P = S // NW
    NCH = P // CH
    NV = D // L
    T = NCH * B
    scale = float(sqrt(D))
    mesh = plsc.VectorSubcoreMesh(core_axis_name="c", subcore_axis_name="s")

    @functools.partial(
        pl.kernel,
        out_type=jax.ShapeDtypeStruct((B * S, D), jnp.float32),
        mesh=mesh,
        scratch_types=[
            pltpu.VMEM((B, P), jnp.int32),
            [pltpu.VMEM((CH, D), jnp.float32) for _ in range(2)],
            [pltpu.VMEM((CH, D), jnp.float32) for _ in range(NBUF)],
            pltpu.SemaphoreType.DMA,
            pltpu.SemaphoreType.DMA,
            pltpu.SemaphoreType.DMA,
        ],
    )
    def k(xf_hbm, table_hbm, pe_hbm, out_hbm, idx_v, pe_v, gbuf, gsem, wsem, psem):
        wid = lax.axis_index("s") * NC + lax.axis_index("c")
        pos_base = wid * P

        for b in range(B):
            pltpu.sync_copy(xf_hbm.at[pl.ds(b * S + pos_base, P)], idx_v.at[b])

        def row_base(t):
            c, b = divmod(t, B)
            return b * S + pos_base + c * CH

        def start_gather(t):
            c, b = divmod(t, B)
            return pltpu.async_copy(
                table_hbm.at[idx_v.at[b, pl.ds(c * CH, CH)]],
                gbuf[t % NBUF], gsem)

        def start_pe(c):
            return pltpu.async_copy(
                pe_hbm.at[pl.ds(pos_base + c * CH, CH)], pe_v[c % 2], psem)

        pe_copies = [start_pe(0)]
        gathers = [start_gather(t) for t in range(LOOKAHEAD)]
        writes = []

        for t in range(T):
            c, b = divmod(t, B)
            if b == 0:
                pe_copies[c].wait()
                if c + 1 < NCH:
                    pe_copies.append(start_pe(c + 1))
            gathers[t].wait()
            g = gbuf[t % NBUF]
            p = pe_v[c % 2]

            def body(j, _, g=g, p=p):
                sl = pl.ds(j * L, L)
                for r in range(CH):
                    g[r, sl] = g[r, sl] * scale + p[r, sl]
                return _

            lax.fori_loop(0, NV, body, 0)
            writes.append(pltpu.async_copy(
                g, out_hbm.at[pl.ds(row_base(t), CH)], wsem))
            if t + LOOKAHEAD < T:
                prev = t + LOOKAHEAD - NBUF
                if prev >= 0:
                    writes[prev].wait()
                gathers.append(start_gather(t + LOOKAHEAD))
        for t in range(max(0, T - NBUF), T):
            writes[t].wait()

    return k


def kernel(x, embed_table, pe):
    B, S = x.shape
    V, D = embed_table.shape
    xf = x.reshape(B * S).astype(jnp.int32)
    pe2 = pe.reshape(pe.shape[-2], pe.shape[-1])[:S]
    out = _build(B, S, V, D)(xf, embed_table, pe2)
    return out.reshape(B, S, D)
